# baseline (speedup 1.0000x reference)
import functools

import jax
import jax.numpy as jnp
from jax import lax
from jax.experimental import pallas as pl
from jax.experimental.pallas import tpu as pltpu
from jax.experimental.pallas import tpu_sc as plsc

EMBED_DIM = 64
BTILE = 128
LANES = 16


@functools.cache
def _make_kernel(batch, hist):
    info = plsc.get_sparse_core_info()
    num_cores, num_subcores = info.num_cores, info.num_subcores
    num_workers = num_cores * num_subcores
    n_btiles = batch // BTILE
    assert n_btiles == num_workers and hist % 2 == 0
    dgroups = EMBED_DIM // 8

    mesh = plsc.VectorSubcoreMesh(core_axis_name="c", subcore_axis_name="s")

    @functools.partial(
        pl.kernel,
        mesh=mesh,
        out_type=jax.ShapeDtypeStruct(
            (hist, dgroups, n_btiles, 8, BTILE), jnp.float32
        ),
        scratch_types=[
            pltpu.VMEM((hist, BTILE), jnp.int32),
            pltpu.VMEM((2, BTILE, EMBED_DIM), jnp.float32),
            pltpu.VMEM((2, dgroups, 8, BTILE + 1), jnp.float32),
        ]
        + [pltpu.SemaphoreType.DMA] * 4,
        compiler_params=pltpu.CompilerParams(
            use_tc_tiling_on_sc=False, needs_layout_passes=False
        ),
    )
    def emb_kernel(xt_hbm, table_hbm, out_hbm, idx_v, rows_v, tr_v, *sems):
        gsem, wsem = sems[:2], sems[2:]
        wid = lax.axis_index("s") * num_cores + lax.axis_index("c")
        pltpu.sync_copy(xt_hbm.at[:, pl.ds(wid * BTILE, BTILE)], idx_v)

        dbase = lax.iota(jnp.int32, LANES)
        dgv = [
            lax.shift_right_logical(dbase + k * LANES, 3)
            for k in range(EMBED_DIM // LANES)
        ]
        dlv = [
            lax.bitwise_and(dbase + k * LANES, 7)
            for k in range(EMBED_DIM // LANES)
        ]

        def gather(h, p):
            return pltpu.make_async_copy(
                table_hbm.at[idx_v.at[h]], rows_v.at[p], gsem[p]
            )

        def wcopy(h, p, dg):
            return pltpu.make_async_copy(
                tr_v.at[p, dg, :, pl.ds(0, BTILE)], out_hbm.at[h, dg, wid], wsem[p]
            )

        def transpose(p):
            @plsc.parallel_loop(0, BTILE, step=2)
            def rbody(b0):
                for i in range(2):
                    b = b0 + i
                    bv = jnp.zeros((LANES,), jnp.int32) + b
                    for k in range(EMBED_DIM // LANES):
                        vals = rows_v[p, b, pl.ds(k * LANES, LANES)]
                        plsc.store_scatter(
                            tr_v.at[p], [dgv[k], dlv[k], bv], vals
                        )

        def write_start(h, p):
            for dg in range(dgroups):
                wcopy(h, p, dg).start()

        def write_wait(h, p):
            for dg in range(dgroups):
                wcopy(h, p, dg).wait()

        n_groups = hist // 2
        gather(0, 0).start()
        gather(1, 1).start()

        def body(g, carry):
            for p in range(2):
                h = 2 * g + p
                gather(h, p).wait()

                @pl.when(g > 0)
                def _():
                    write_wait(h - 2, p)

                transpose(p)
                write_start(h, p)

                @pl.when(g < n_groups - 1)
                def _():
                    gather(h + 2, p).start()

            return carry

        lax.fori_loop(0, n_groups, body, 0)

        for p in range(2):
            write_wait(hist - 2 + p, p)

    return emb_kernel


def kernel(x, table):
    batch, hist = x.shape
    out5 = _make_kernel(batch, hist)(x.T, table)
    return jnp.transpose(out5, (2, 4, 0, 1, 3)).reshape(batch, hist, EMBED_DIM)
